# baseline (speedup 1.0000x reference)
import functools

import jax
import jax.numpy as jnp
from jax import lax
from jax.experimental import pallas as pl
from jax.experimental.pallas import tpu as pltpu
from jax.experimental.pallas import tpu_sc as plsc

_BATCH = 4096
_ATOMS = 50
_N = _BATCH * _ATOMS
_NUM_WORKERS = 32
_CHUNK = _N // _NUM_WORKERS
_LANES = 16
_TABLE_PAD = 128
_GROUP = 64
_WCHUNK = _CHUNK // 4


def _sc_body(energies_hbm, table_hbm, idxw_hbm, out_hbm, table_v, idxw_v, e_v, sems):
    wid = lax.axis_index("s") * 2 + lax.axis_index("c")
    base = wid * _CHUNK
    ct = pltpu.async_copy(table_hbm, table_v, sems.at[0])
    ci = pltpu.async_copy(idxw_hbm.at[pl.ds(wid * _WCHUNK, _WCHUNK)], idxw_v, sems.at[1])
    ce = pltpu.async_copy(energies_hbm.at[pl.ds(base, _CHUNK)], e_v, sems.at[2])
    ct.wait()
    ci.wait()
    ce.wait()

    @plsc.parallel_loop(0, _CHUNK // _GROUP, step=1, unroll=4)
    def _gather_add(gi):
        words = idxw_v[pl.ds(gi * _LANES, _LANES)]
        for k in range(4):
            idxk = (words >> (8 * k)) & 0xFF
            vals = plsc.load_gather(table_v, [idxk])
            plsc.addupdate(e_v.at[pl.ds(gi * _GROUP + k * _LANES, _LANES)], vals)

    pltpu.sync_copy(e_v, out_hbm.at[pl.ds(base, _CHUNK)])


@jax.jit
def _run(energies_flat, table_pad, idx_flat):
    mesh = plsc.VectorSubcoreMesh(core_axis_name="c", subcore_axis_name="s")
    fn = functools.partial(
        pl.kernel,
        mesh=mesh,
        out_type=jax.ShapeDtypeStruct((_N,), jnp.float32),
        scratch_types=[
            pltpu.VMEM((_TABLE_PAD,), jnp.float32),
            pltpu.VMEM((_WCHUNK,), jnp.int32),
            pltpu.VMEM((_CHUNK,), jnp.float32),
            pltpu.SemaphoreType.DMA((3,)),
        ],
        compiler_params=pltpu.CompilerParams(needs_layout_passes=False),
    )(_sc_body)
    return fn(energies_flat, table_pad, idx_flat)


def kernel(atomic_energies, atom_ref, atomic_numbers):
    energies_flat = atomic_energies.reshape(_N)
    idx_words = jax.lax.bitcast_convert_type(
        atomic_numbers.reshape(_N // _GROUP, 4, _LANES)
        .swapaxes(1, 2).astype(jnp.int8), jnp.int32).reshape(_N // 4)
    table_pad = jnp.pad(atom_ref.reshape(-1), (0, _TABLE_PAD - atom_ref.shape[0]))
    out = _run(energies_flat, table_pad, idx_words)
    return out.reshape(_BATCH, _ATOMS, 1)


# speedup vs baseline: 1.7012x; 1.7012x over previous
import functools

import jax
import jax.numpy as jnp
from jax import lax
from jax.experimental import pallas as pl
from jax.experimental.pallas import tpu as pltpu
from jax.experimental.pallas import tpu_sc as plsc

_BATCH = 4096
_ATOMS = 50
_N = _BATCH * _ATOMS
_NUM_WORKERS = 16
_CHUNK = _N // _NUM_WORKERS
_LANES = 16
_TABLE_PAD = 128


def _sc_body(energies_hbm, table_hbm, idx_hbm, out_hbm, table_v, idx_v, e_v, sems):
    wid = lax.axis_index("s")
    base = wid * _CHUNK
    ct = pltpu.async_copy(table_hbm, table_v, sems.at[0])
    ci = pltpu.async_copy(idx_hbm.at[pl.ds(base, _CHUNK)], idx_v, sems.at[1])
    ce = pltpu.async_copy(energies_hbm.at[pl.ds(base, _CHUNK)], e_v, sems.at[2])
    ct.wait()
    ci.wait()
    ce.wait()

    @plsc.parallel_loop(0, _CHUNK, step=_LANES, unroll=8)
    def _gather_add(i):
        sl = pl.ds(i, _LANES)
        vals = plsc.load_gather(table_v, [idx_v[sl]])
        plsc.addupdate(e_v.at[sl], vals)

    pltpu.sync_copy(e_v, out_hbm.at[pl.ds(base, _CHUNK)])


@jax.jit
def _run(energies_flat, table_pad, idx_flat):
    mesh = plsc.VectorSubcoreMesh(core_axis_name="c", subcore_axis_name="s", num_cores=1)
    fn = functools.partial(
        pl.kernel,
        mesh=mesh,
        out_type=jax.ShapeDtypeStruct((_N,), jnp.float32),
        scratch_types=[
            pltpu.VMEM((_TABLE_PAD,), jnp.float32),
            pltpu.VMEM((_CHUNK,), jnp.int32),
            pltpu.VMEM((_CHUNK,), jnp.float32),
            pltpu.SemaphoreType.DMA((3,)),
        ],
        compiler_params=pltpu.CompilerParams(needs_layout_passes=False),
    )(_sc_body)
    return fn(energies_flat, table_pad, idx_flat)


def kernel(atomic_energies, atom_ref, atomic_numbers):
    energies_flat = atomic_energies.reshape(_N)
    idx_flat = atomic_numbers.reshape(_N).astype(jnp.int32)
    table_pad = jnp.pad(atom_ref.reshape(-1), (0, _TABLE_PAD - atom_ref.shape[0]))
    out = _run(energies_flat, table_pad, idx_flat)
    return out.reshape(_BATCH, _ATOMS, 1)
